# Initial kernel scaffold; baseline (speedup 1.0000x reference)
import functools

import jax
import jax.numpy as jnp
from jax import lax
from jax.experimental import pallas as pl
from jax.experimental.pallas import tpu as pltpu
from jax.experimental.pallas import tpu_sc as plsc

BOS = 0
CHUNK = 128


@functools.lru_cache(maxsize=None)
def _build(batch: int, seq: int, vocab: int, d_model: int):
    info = plsc.get_sparse_core_info()
    num_workers = info.num_cores * info.num_subcores

    total = batch * seq
    assert total % num_workers == 0
    per_w = total // num_workers
    assert per_w % CHUNK == 0
    n_chunks = per_w // CHUNK
    workers_per_batch = num_workers // batch
    out_rows = batch * (seq + 1)

    mesh = plsc.VectorSubcoreMesh(core_axis_name="c", subcore_axis_name="s")

    @functools.partial(
        pl.kernel,
        mesh=mesh,
        out_type=jax.ShapeDtypeStruct((out_rows, d_model), jnp.float32),
        scratch_types=[
            pltpu.VMEM((per_w,), jnp.int32),
            pltpu.VMEM((per_w, d_model), jnp.float32),
            pltpu.VMEM((1, d_model), jnp.float32),
            pltpu.SemaphoreType.DMA,
        ],
    )
    def emb(ids_hbm, table_hbm, out_hbm, idx_v, rows_v, bos_v, sem):
        wid = lax.axis_index("s") * info.num_cores + lax.axis_index("c")
        b = wid // workers_per_batch
        sub = wid % workers_per_batch
        in_base = wid * per_w
        out_base = b * (seq + 1) + 1 + sub * per_w

        pltpu.sync_copy(ids_hbm.at[pl.ds(in_base, per_w)], idx_v)

        copies = []
        for j in range(n_chunks):
            copies.append(
                pltpu.async_copy(
                    table_hbm.at[idx_v.at[pl.ds(j * CHUNK, CHUNK)]],
                    rows_v.at[pl.ds(j * CHUNK, CHUNK)],
                    sem,
                )
            )

        @pl.when(sub == 0)
        def _write_bos():
            pltpu.sync_copy(table_hbm.at[pl.ds(BOS, 1)], bos_v)
            pltpu.sync_copy(bos_v, out_hbm.at[pl.ds(b * (seq + 1), 1)])

        for c in copies:
            c.wait()
        pltpu.sync_copy(rows_v, out_hbm.at[pl.ds(out_base, per_w)])

    return emb


def kernel(input_ids, embedding_table):
    batch, seq = input_ids.shape
    vocab, d_model = embedding_table.shape
    flat_ids = input_ids.reshape(-1).astype(jnp.int32)
    emb = _build(batch, seq, vocab, d_model)
    out = emb(flat_ids, embedding_table)
    return out.reshape(batch, seq + 1, d_model)


# speedup vs baseline: 1.0010x; 1.0010x over previous
import functools

import jax
import jax.numpy as jnp
from jax import lax
from jax.experimental import pallas as pl
from jax.experimental.pallas import tpu as pltpu
from jax.experimental.pallas import tpu_sc as plsc

BOS = 0
CHUNK = 128


@functools.lru_cache(maxsize=None)
def _build(n_ids: int, vocab: int, d_model: int):
    info = plsc.get_sparse_core_info()
    num_workers = info.num_cores * info.num_subcores

    per_w = (n_ids // num_workers) // 8 * 8
    n_chunks = per_w // CHUNK
    assert per_w % CHUNK == 0 and n_chunks >= 1
    tail_base = per_w * num_workers
    tail = n_ids - tail_base
    assert 0 <= tail <= CHUNK

    mesh = plsc.VectorSubcoreMesh(core_axis_name="c", subcore_axis_name="s")

    scratch = [
        pltpu.VMEM((per_w,), jnp.int32),
        pltpu.VMEM((per_w, d_model), jnp.float32),
        pltpu.SemaphoreType.DMA,
    ]
    if tail:
        scratch += [
            pltpu.VMEM((tail,), jnp.int32),
            pltpu.VMEM((tail, d_model), jnp.float32),
            pltpu.SemaphoreType.DMA,
        ]

    @functools.partial(
        pl.kernel,
        mesh=mesh,
        out_type=jax.ShapeDtypeStruct((n_ids, d_model), jnp.float32),
        scratch_types=scratch,
        compiler_params=pltpu.CompilerParams(use_tc_tiling_on_sc=False),
    )
    def emb(ids_hbm, table_hbm, out_hbm, idx_v, rows_v, sem, *tail_scratch):
        wid = lax.axis_index("s") * info.num_cores + lax.axis_index("c")
        base = wid * per_w

        pltpu.sync_copy(ids_hbm.at[pl.ds(base, per_w)], idx_v)

        copies = []
        for j in range(n_chunks):
            copies.append(
                pltpu.async_copy(
                    table_hbm.at[idx_v.at[pl.ds(j * CHUNK, CHUNK)]],
                    rows_v.at[pl.ds(j * CHUNK, CHUNK)],
                    sem,
                )
            )

        if tail:
            tidx_v, trows_v, tsem = tail_scratch

            @pl.when(wid == num_workers - 1)
            def _do_tail():
                pltpu.sync_copy(ids_hbm.at[pl.ds(tail_base, tail)], tidx_v)
                pltpu.async_copy(table_hbm.at[tidx_v], trows_v, tsem).wait()
                pltpu.sync_copy(trows_v, out_hbm.at[pl.ds(tail_base, tail)])

        for c in copies:
            c.wait()
        pltpu.sync_copy(rows_v, out_hbm.at[pl.ds(base, per_w)])

    return emb


def kernel(input_ids, embedding_table):
    batch, seq = input_ids.shape
    vocab, d_model = embedding_table.shape
    padded = jnp.pad(input_ids, ((0, 0), (1, 0)), constant_values=BOS)
    flat_ids = padded.reshape(-1).astype(jnp.int32)
    emb = _build(flat_ids.shape[0], vocab, d_model)
    out = emb(flat_ids, embedding_table)
    return out.reshape(batch, seq + 1, d_model)
